# baseline (speedup 1.0000x reference)
import functools

import jax
import jax.numpy as jnp
from jax import lax
from jax.experimental import pallas as pl
from jax.experimental.pallas import tpu as pltpu
from jax.experimental.pallas import tpu_sc as plsc

D_MODEL = 64
NUM_CORES = 2
NUM_SUBCORES = 16
NUM_WORKERS = NUM_CORES * NUM_SUBCORES
CHUNK = 1024
DEPTH = 16


def kernel(x, table):
    batch, seq = x.shape
    n = batch * seq
    idx_flat = x.astype(jnp.int32).reshape(n)

    per_worker = n // NUM_WORKERS
    mesh = plsc.VectorSubcoreMesh(core_axis_name="c", subcore_axis_name="s")

    @functools.partial(
        pl.kernel,
        mesh=mesh,
        out_type=jax.ShapeDtypeStruct((n, D_MODEL), table.dtype),
        scratch_types=[
            pltpu.VMEM((CHUNK,), jnp.int32),
            pltpu.SemaphoreType.DMA,
        ],
    )
    def gather_kernel(table_hbm, idx_hbm, out_hbm, idx_v, sem):
        wid = lax.axis_index("s") * NUM_CORES + lax.axis_index("c")
        base = wid * per_worker

        def row_copy(src_row, dst_row):
            return pltpu.make_async_copy(
                table_hbm.at[pl.ds(src_row, 1)],
                out_hbm.at[pl.ds(dst_row, 1)],
                sem,
            )

        @pl.loop(0, per_worker, step=CHUNK)
        def _(off):
            pltpu.sync_copy(idx_hbm.at[pl.ds(base + off, CHUNK)], idx_v)

            @pl.loop(0, CHUNK, step=16)
            def _(i):
                v = idx_v[pl.ds(i, 16)]
                for j in range(16):
                    row_copy(v[j], base + off + i + j).start()

                @pl.when(i >= DEPTH)
                def _():
                    for _j in range(16):
                        row_copy(0, 0).wait()

            @pl.loop(0, DEPTH)
            def _(i):
                row_copy(0, 0).wait()

    out_flat = gather_kernel(table, idx_flat)
    return out_flat.reshape(batch, seq, D_MODEL)
